# baseline (speedup 1.0000x reference)
import functools

import jax
import jax.numpy as jnp
from jax import lax
from jax.experimental import pallas as pl
from jax.experimental.pallas import tpu as pltpu
from jax.experimental.pallas import tpu_sc as plsc

N_NODES, N_EDGES, D_IN, HID, WIDTH, D_OUT = 10000, 320000, 128, 64, 3, 128
F_MID = HID * WIDTH
F_WIDE = D_OUT * WIDTH

NCORES, NSUB = 2, 16
NPAD = 10240
ROWS_PER_TILE = NPAD // NSUB
BLK = 128
NBLK = 160
EDGES_PER_TILE = NBLK * BLK
E_PAD = NSUB * EDGES_PER_TILE
KGRP = 1

TC_ROWS = 512


@functools.lru_cache(maxsize=None)
def _make_sc_aggr(fpart, with_deg, nbuf):
  mesh = plsc.VectorSubcoreMesh(core_axis_name="c", subcore_axis_name="s",
                                num_cores=NCORES, num_subcores=NSUB)

  del nbuf
  out_type = [jax.ShapeDtypeStruct((NCORES, NPAD, fpart), jnp.float32)]
  scratch = [
      pltpu.VMEM((NBLK, BLK), jnp.int32),
      pltpu.VMEM((NBLK, BLK), jnp.int32),
      [pltpu.VMEM((BLK, fpart), jnp.float32) for _ in range(2 * KGRP)],
      pltpu.VMEM_SHARED((NPAD, fpart), jnp.float32),
      pltpu.SemaphoreType.DMA,
      pltpu.SemaphoreType.DMA,
  ]
  if with_deg:
    out_type.append(jax.ShapeDtypeStruct((NCORES, NPAD), jnp.float32))
    scratch.append(pltpu.VMEM((BLK,), jnp.float32))
    scratch.append(pltpu.VMEM_SHARED((NPAD,), jnp.float32))
    scratch.append(pltpu.SemaphoreType.DMA)

  def body(*refs):
    if with_deg:
      (x_hbm, src_hbm, dst_hbm, zrows_hbm, z1_hbm,
       acc_out, deg_out,
       src_v, dst_v, rows_v, acc_sh, gsem, ssem,
       ones_v, deg_sh, dsem) = refs
    else:
      (x_hbm, src_hbm, dst_hbm, zrows_hbm,
       acc_out,
       src_v, dst_v, rows_v, acc_sh, gsem, ssem) = refs

    c = lax.axis_index("c")
    s = lax.axis_index("s")
    z0 = s * ROWS_PER_TILE

    def gather(j, buf):
      return pltpu.make_async_copy(x_hbm.at[c].at[src_v.at[j]],
                                   rows_v[buf], gsem)

    pltpu.sync_copy(src_hbm.at[s], src_v)
    pltpu.sync_copy(dst_hbm.at[s], dst_v)
    for i in range(KGRP):
      gather(i, i).start()

    pltpu.sync_copy(zrows_hbm, acc_sh.at[pl.ds(z0, ROWS_PER_TILE)])
    if with_deg:
      pltpu.sync_copy(z1_hbm, deg_sh.at[pl.ds(z0, ROWS_PER_TILE)])
      for k in range(BLK // 16):
        ones_v[pl.ds(k * 16, 16)] = jnp.ones((16,), jnp.float32)
      do_deg = c == 0

    plsc.subcore_barrier()

    def scat_drain(buf):
      pltpu.make_async_copy(rows_v[buf], acc_sh.at[dst_v.at[0]], ssem).wait()

    def deg_drain():
      @pl.when(do_deg)
      def _():
        for _i in range(KGRP):
          pltpu.make_async_copy(ones_v, deg_sh.at[dst_v.at[0]], dsem).wait()

    def phase(p, grp, drain_prev, issue_next):
      o = 1 - grp
      if drain_prev:
        for i in range(KGRP):
          scat_drain(KGRP * o + i)
        if with_deg:
          deg_drain()
      for i in range(KGRP):
        gather(p * KGRP + i, KGRP * grp + i).wait()
      for i in range(KGRP):
        pltpu.async_copy(rows_v[KGRP * grp + i],
                         acc_sh.at[dst_v.at[p * KGRP + i]], ssem, add=True)
      if with_deg:
        @pl.when(do_deg)
        def _():
          for i in range(KGRP):
            pltpu.async_copy(ones_v, deg_sh.at[dst_v.at[p * KGRP + i]],
                             dsem, add=True)
      if issue_next:
        for i in range(KGRP):
          gather((p + 1) * KGRP + i, KGRP * o + i).start()

    n_phases = NBLK // KGRP
    phase(0, 0, drain_prev=False, issue_next=True)

    def super_fn(g, carry):
      phase(2 * g + 1, 1, True, True)
      phase(2 * g + 2, 0, True, True)
      return carry

    lax.fori_loop(0, (n_phases - 2) // 2, super_fn, 0)

    phase(n_phases - 1, 1, drain_prev=True, issue_next=False)
    for i in range(KGRP):
      scat_drain(KGRP * 1 + i)
    if with_deg:
      deg_drain()

    plsc.subcore_barrier()
    pltpu.sync_copy(acc_sh.at[pl.ds(z0, ROWS_PER_TILE)],
                    acc_out.at[c, pl.ds(z0, ROWS_PER_TILE)])
    if with_deg:
      pltpu.sync_copy(deg_sh.at[pl.ds(z0, ROWS_PER_TILE)],
                      deg_out.at[c, pl.ds(z0, ROWS_PER_TILE)])

  return pl.kernel(body, out_type=tuple(out_type), mesh=mesh,
                   scratch_types=scratch,
                   compiler_params=pltpu.CompilerParams(
                       use_tc_tiling_on_sc=False))


def _tc_layer_body(split_in, pool, *refs):
  if split_in:
    x_ref, a_ref, d_ref, data_ref, w_ref, wt_ref, bt_ref, *rest = refs
    x = jnp.concatenate([x_ref[0], x_ref[1]], axis=1)
  else:
    a_ref, d_ref, data_ref, w_ref, wt_ref, bt_ref, *rest = refs
    x = data_ref[...]
  if pool:
    p_ref, o_ref = rest
  else:
    (o_ref,) = rest
  acc = jnp.concatenate([a_ref[0], a_ref[1]], axis=1)
  deg = d_ref[...]
  has = deg > 0.0
  scale = jnp.where(has, 1.0 / jnp.where(has, deg, 1.0), 0.0)
  lap = jnp.where(has, x, 0.0) - scale * acc
  h = jnp.dot(lap, w_ref[...], preferred_element_type=jnp.float32)
  h = h + jnp.dot(data_ref[...], wt_ref[...],
                  preferred_element_type=jnp.float32)
  h = jnp.maximum(h + bt_ref[...], 0.0)
  if pool:
    o_ref[...] = jnp.dot(h, p_ref[...], preferred_element_type=jnp.float32)
  else:
    half = h.shape[1] // 2
    o_ref[0] = h[:, :half]
    o_ref[1] = h[:, half:]


def _make_tc_layer(f_in, f_out, split_in, pool_cols=None):
  grid = (NPAD // TC_ROWS,)
  part = lambda f: pl.BlockSpec((NCORES, TC_ROWS, f // 2),
                                lambda i: (0, i, 0))
  full = lambda a, b: pl.BlockSpec((a, b), lambda i: (0, 0))
  in_specs = []
  if split_in:
    in_specs.append(part(f_in))
  in_specs += [
      part(f_in),
      pl.BlockSpec((TC_ROWS, 1), lambda i: (i, 0)),
      pl.BlockSpec((TC_ROWS, D_IN), lambda i: (i, 0)),
      full(f_in, f_out),
      full(D_IN, f_out),
      full(1, f_out),
  ]
  if pool_cols is not None:
    in_specs.append(full(f_out, pool_cols))
    out_spec = pl.BlockSpec((TC_ROWS, pool_cols), lambda i: (i, 0))
    out_shape = jax.ShapeDtypeStruct((NPAD, pool_cols), jnp.float32)
  else:
    out_spec = pl.BlockSpec((NCORES, TC_ROWS, f_out // 2),
                            lambda i: (0, i, 0))
    out_shape = jax.ShapeDtypeStruct((NCORES, NPAD, f_out // 2), jnp.float32)
  return pl.pallas_call(
      functools.partial(_tc_layer_body, split_in, pool_cols is not None),
      grid=grid,
      in_specs=in_specs,
      out_specs=out_spec,
      out_shape=out_shape,
  )


_tc_layer1 = _make_tc_layer(D_IN, F_MID, split_in=False)
_tc_layer2 = _make_tc_layer(F_MID, F_MID, split_in=True)
_tc_layer3 = _make_tc_layer(F_MID, F_WIDE, split_in=True, pool_cols=D_OUT)


def kernel(data, structure, pre_W, pre_b, pre_Wt, pre_bt,
           blk_Wg, blk_bg, blk_Wt, blk_bt,
           post_Wg, post_bg, post_Wt, post_bt):
  del pre_b, blk_bg, post_bg
  n_fake = E_PAD - N_EDGES
  fake = N_NODES + (jnp.arange(n_fake, dtype=jnp.int32)
                    % (NPAD - N_NODES))
  src = jnp.concatenate([structure[0], fake]).reshape(NSUB, NBLK, BLK)
  dst = jnp.concatenate([structure[1], fake]).reshape(NSUB, NBLK, BLK)
  x0 = jnp.zeros((NPAD, D_IN), jnp.float32).at[:N_NODES].set(data)
  x0_parts = x0.reshape(NPAD, NCORES, D_IN // 2).transpose(1, 0, 2)

  zrows_in = jnp.zeros((ROWS_PER_TILE, D_IN // 2), jnp.float32)
  zrows_mid = jnp.zeros((ROWS_PER_TILE, F_MID // 2), jnp.float32)
  z1 = jnp.zeros((ROWS_PER_TILE,), jnp.float32)

  w2 = jax.scipy.linalg.block_diag(*[blk_Wg[g] for g in range(WIDTH)])
  w3 = jax.scipy.linalg.block_diag(*[post_Wg[g] for g in range(WIDTH)])
  pool = ((jnp.arange(F_WIDE)[:, None] // WIDTH)
          == jnp.arange(D_OUT)[None, :]).astype(jnp.float32) / WIDTH

  acc1, deg = _make_sc_aggr(D_IN // 2, True, 4)(x0_parts, src, dst,
                                                zrows_in, z1)
  d0 = deg[0].reshape(NPAD, 1)

  out1 = _tc_layer1(acc1, d0, x0,
                    pre_W, pre_Wt, pre_bt.reshape(1, F_MID))

  (acc2,) = _make_sc_aggr(F_MID // 2, False, 2)(out1, src, dst, zrows_mid)
  out2 = _tc_layer2(out1, acc2, d0, x0,
                    w2, blk_Wt, blk_bt.reshape(1, F_MID))

  (acc3,) = _make_sc_aggr(F_MID // 2, False, 2)(out2, src, dst, zrows_mid)
  y = _tc_layer3(out2, acc3, d0, x0,
                 w3, post_Wt, post_bt.reshape(1, F_WIDE), pool)

  return y[:N_NODES].reshape(N_NODES, D_OUT, 1)


# speedup vs baseline: 1.2722x; 1.2722x over previous
import functools

import jax
import jax.numpy as jnp
from jax import lax
from jax.experimental import pallas as pl
from jax.experimental.pallas import tpu as pltpu
from jax.experimental.pallas import tpu_sc as plsc

N_NODES, N_EDGES, D_IN, HID, WIDTH, D_OUT = 10000, 320000, 128, 64, 3, 128
F_MID = HID * WIDTH
F_WIDE = D_OUT * WIDTH

NCORES, NSUB = 2, 16
NPAD = 10240
ROWS_PER_TILE = NPAD // NSUB
BLK = 128
NBLK = 160
EDGES_PER_TILE = NBLK * BLK
E_PAD = NSUB * EDGES_PER_TILE

TC_ROWS = 512


@functools.lru_cache(maxsize=None)
def _make_sc_aggr(fpart, with_deg, nbuf):
  mesh = plsc.VectorSubcoreMesh(core_axis_name="c", subcore_axis_name="s",
                                num_cores=NCORES, num_subcores=NSUB)

  out_type = [jax.ShapeDtypeStruct((NCORES, NPAD, fpart), jnp.float32)]
  scratch = [
      pltpu.VMEM((NBLK, BLK), jnp.int32),
      pltpu.VMEM((NBLK, BLK), jnp.int32),
      [pltpu.VMEM((BLK, fpart), jnp.float32) for _ in range(nbuf)],
      pltpu.VMEM_SHARED((NPAD, fpart), jnp.float32),
      [pltpu.SemaphoreType.DMA for _ in range(nbuf)],
  ]
  if with_deg:
    out_type.append(jax.ShapeDtypeStruct((NCORES, NPAD), jnp.float32))
    scratch.append(pltpu.VMEM((BLK,), jnp.float32))
    scratch.append(pltpu.VMEM_SHARED((NPAD,), jnp.float32))

  def body(*refs):
    if with_deg:
      (x_hbm, src_hbm, dst_hbm, zrows_hbm, z1_hbm,
       acc_out, deg_out,
       src_v, dst_v, rows_v, acc_sh, gsem, ones_v, deg_sh) = refs
    else:
      (x_hbm, src_hbm, dst_hbm, zrows_hbm,
       acc_out,
       src_v, dst_v, rows_v, acc_sh, gsem) = refs

    c = lax.axis_index("c")
    s = lax.axis_index("s")
    z0 = s * ROWS_PER_TILE

    def gather(j, b):
      return pltpu.make_async_copy(x_hbm.at[c].at[src_v.at[j]],
                                   rows_v[b], gsem[b])

    pltpu.sync_copy(src_hbm.at[s], src_v)
    pltpu.sync_copy(dst_hbm.at[s], dst_v)
    for b in range(nbuf):
      gather(b, b).start()

    pltpu.sync_copy(zrows_hbm, acc_sh.at[pl.ds(z0, ROWS_PER_TILE)])
    if with_deg:
      pltpu.sync_copy(z1_hbm, deg_sh.at[pl.ds(z0, ROWS_PER_TILE)])
      for k in range(BLK // 16):
        ones_v[pl.ds(k * 16, 16)] = jnp.ones((16,), jnp.float32)
      do_deg = c == 0

    plsc.subcore_barrier()

    n_rounds = NBLK // nbuf

    def round_fn(g, carry):
      for b in range(nbuf):
        j = g * nbuf + b
        gather(j, b).wait()
        pltpu.sync_copy(rows_v[b], acc_sh.at[dst_v.at[j]], add=True)
        if with_deg:
          @pl.when(do_deg)
          def _():
            pltpu.sync_copy(ones_v, deg_sh.at[dst_v.at[j]], add=True)

        @pl.when(g + 1 < n_rounds)
        def _():
          gather(j + nbuf, b).start()
      return carry

    lax.fori_loop(0, n_rounds, round_fn, 0)

    plsc.subcore_barrier()
    pltpu.sync_copy(acc_sh.at[pl.ds(z0, ROWS_PER_TILE)],
                    acc_out.at[c, pl.ds(z0, ROWS_PER_TILE)])
    if with_deg:
      pltpu.sync_copy(deg_sh.at[pl.ds(z0, ROWS_PER_TILE)],
                      deg_out.at[c, pl.ds(z0, ROWS_PER_TILE)])

  return pl.kernel(body, out_type=tuple(out_type), mesh=mesh,
                   scratch_types=scratch,
                   compiler_params=pltpu.CompilerParams(
                       use_tc_tiling_on_sc=False))


def _tc_layer_body(split_in, pool, *refs):
  if split_in:
    x_ref, a_ref, d_ref, data_ref, w_ref, wt_ref, bt_ref, *rest = refs
    x = jnp.concatenate([x_ref[0], x_ref[1]], axis=1)
  else:
    a_ref, d_ref, data_ref, w_ref, wt_ref, bt_ref, *rest = refs
    x = data_ref[...]
  if pool:
    p_ref, o_ref = rest
  else:
    (o_ref,) = rest
  acc = jnp.concatenate([a_ref[0], a_ref[1]], axis=1)
  deg = d_ref[...]
  has = deg > 0.0
  scale = jnp.where(has, 1.0 / jnp.where(has, deg, 1.0), 0.0)
  lap = jnp.where(has, x, 0.0) - scale * acc
  h = jnp.dot(lap, w_ref[...], preferred_element_type=jnp.float32)
  h = h + jnp.dot(data_ref[...], wt_ref[...],
                  preferred_element_type=jnp.float32)
  h = jnp.maximum(h + bt_ref[...], 0.0)
  if pool:
    o_ref[...] = jnp.dot(h, p_ref[...], preferred_element_type=jnp.float32)
  else:
    half = h.shape[1] // 2
    o_ref[0] = h[:, :half]
    o_ref[1] = h[:, half:]


def _make_tc_layer(f_in, f_out, split_in, pool_cols=None):
  grid = (NPAD // TC_ROWS,)
  part = lambda f: pl.BlockSpec((NCORES, TC_ROWS, f // 2),
                                lambda i: (0, i, 0))
  full = lambda a, b: pl.BlockSpec((a, b), lambda i: (0, 0))
  in_specs = []
  if split_in:
    in_specs.append(part(f_in))
  in_specs += [
      part(f_in),
      pl.BlockSpec((TC_ROWS, 1), lambda i: (i, 0)),
      pl.BlockSpec((TC_ROWS, D_IN), lambda i: (i, 0)),
      full(f_in, f_out),
      full(D_IN, f_out),
      full(1, f_out),
  ]
  if pool_cols is not None:
    in_specs.append(full(f_out, pool_cols))
    out_spec = pl.BlockSpec((TC_ROWS, pool_cols), lambda i: (i, 0))
    out_shape = jax.ShapeDtypeStruct((NPAD, pool_cols), jnp.float32)
  else:
    out_spec = pl.BlockSpec((NCORES, TC_ROWS, f_out // 2),
                            lambda i: (0, i, 0))
    out_shape = jax.ShapeDtypeStruct((NCORES, NPAD, f_out // 2), jnp.float32)
  return pl.pallas_call(
      functools.partial(_tc_layer_body, split_in, pool_cols is not None),
      grid=grid,
      in_specs=in_specs,
      out_specs=out_spec,
      out_shape=out_shape,
  )


_tc_layer1 = _make_tc_layer(D_IN, F_MID, split_in=False)
_tc_layer2 = _make_tc_layer(F_MID, F_MID, split_in=True)
_tc_layer3 = _make_tc_layer(F_MID, F_WIDE, split_in=True, pool_cols=D_OUT)


def kernel(data, structure, pre_W, pre_b, pre_Wt, pre_bt,
           blk_Wg, blk_bg, blk_Wt, blk_bt,
           post_Wg, post_bg, post_Wt, post_bt):
  del pre_b, blk_bg, post_bg
  n_fake = E_PAD - N_EDGES
  fake = N_NODES + (jnp.arange(n_fake, dtype=jnp.int32)
                    % (NPAD - N_NODES))
  src = jnp.concatenate([structure[0], fake]).reshape(NSUB, NBLK, BLK)
  dst = jnp.concatenate([structure[1], fake]).reshape(NSUB, NBLK, BLK)
  x0 = jnp.zeros((NPAD, D_IN), jnp.float32).at[:N_NODES].set(data)
  x0_parts = x0.reshape(NPAD, NCORES, D_IN // 2).transpose(1, 0, 2)

  zrows_in = jnp.zeros((ROWS_PER_TILE, D_IN // 2), jnp.float32)
  zrows_mid = jnp.zeros((ROWS_PER_TILE, F_MID // 2), jnp.float32)
  z1 = jnp.zeros((ROWS_PER_TILE,), jnp.float32)

  w2 = jax.scipy.linalg.block_diag(*[blk_Wg[g] for g in range(WIDTH)])
  w3 = jax.scipy.linalg.block_diag(*[post_Wg[g] for g in range(WIDTH)])
  pool = ((jnp.arange(F_WIDE)[:, None] // WIDTH)
          == jnp.arange(D_OUT)[None, :]).astype(jnp.float32) / WIDTH

  acc1, deg = _make_sc_aggr(D_IN // 2, True, 4)(x0_parts, src, dst,
                                                zrows_in, z1)
  d0 = deg[0].reshape(NPAD, 1)

  out1 = _tc_layer1(acc1, d0, x0,
                    pre_W, pre_Wt, pre_bt.reshape(1, F_MID))

  (acc2,) = _make_sc_aggr(F_MID // 2, False, 2)(out1, src, dst, zrows_mid)
  out2 = _tc_layer2(out1, acc2, d0, x0,
                    w2, blk_Wt, blk_bt.reshape(1, F_MID))

  (acc3,) = _make_sc_aggr(F_MID // 2, False, 2)(out2, src, dst, zrows_mid)
  y = _tc_layer3(out2, acc3, d0, x0,
                 w3, post_Wt, post_bt.reshape(1, F_WIDE), pool)

  return y[:N_NODES].reshape(N_NODES, D_OUT, 1)
